# Initial kernel scaffold; baseline (speedup 1.0000x reference)
import jax
import jax.numpy as jnp
from jax.experimental import pallas as pl


def kernel(idx, logits_table):
    raise NotImplementedError("write your pallas kernel here")


# speedup vs baseline: 1.3300x; 1.3300x over previous
import functools

import jax
import jax.numpy as jnp
from jax import lax
from jax.experimental import pallas as pl
from jax.experimental.pallas import tpu as pltpu
from jax.experimental.pallas import tpu_sc as plsc

_NC = 2
_NS = 16
_NW = _NC * _NS


def kernel(idx, logits_table):
    B, L = idx.shape
    V, D = logits_table.shape
    N = B * L
    n_per_w = N // _NW
    C = 64
    n_chunks = n_per_w // C

    idx_flat = idx.reshape(_NW, n_chunks, C).astype(jnp.int32)

    mesh = plsc.VectorSubcoreMesh(core_axis_name="c", subcore_axis_name="s")

    @functools.partial(
        pl.kernel,
        mesh=mesh,
        compiler_params=pltpu.CompilerParams(use_tc_tiling_on_sc=False),
        out_type=jax.ShapeDtypeStruct((N, D), jnp.float32),
        scratch_types=[
            pltpu.VMEM((n_chunks, C), jnp.int32),
            pltpu.VMEM((C, D), jnp.float32),
            pltpu.SemaphoreType.DMA,
        ],
    )
    def gather_rows(idx_hbm, table_hbm, out_hbm, idx_v, buf, sem):
        wid = lax.axis_index("s") * _NC + lax.axis_index("c")
        base = wid * n_per_w
        pltpu.sync_copy(idx_hbm.at[wid], idx_v)
        for g in range(n_chunks):
            pltpu.async_copy(table_hbm.at[idx_v.at[g]], buf, sem).wait()
            pltpu.sync_copy(buf, out_hbm.at[pl.ds(base + g * C, C)])

    out = gather_rows(idx_flat, logits_table)
    return out.reshape(B, L, V)


# speedup vs baseline: 1.3593x; 1.0221x over previous
import functools

import jax
import jax.numpy as jnp
from jax import lax
from jax.experimental import pallas as pl
from jax.experimental.pallas import tpu as pltpu
from jax.experimental.pallas import tpu_sc as plsc

_NC = 2
_NS = 16
_NW = _NC * _NS


def kernel(idx, logits_table):
    B, L = idx.shape
    V, D = logits_table.shape
    N = B * L
    n_per_w = N // _NW
    C = 64
    n_chunks = n_per_w // C

    idx_flat = idx.reshape(_NW, n_chunks, C).astype(jnp.int32)

    mesh = plsc.VectorSubcoreMesh(core_axis_name="c", subcore_axis_name="s")

    @functools.partial(
        pl.kernel,
        mesh=mesh,
        compiler_params=pltpu.CompilerParams(use_tc_tiling_on_sc=False),
        out_type=jax.ShapeDtypeStruct((N, D), jnp.float32),
        scratch_types=[
            pltpu.VMEM((n_chunks, C), jnp.int32),
            pltpu.VMEM((C, D), jnp.float32),
            pltpu.VMEM((C, D), jnp.float32),
            pltpu.SemaphoreType.DMA,
            pltpu.SemaphoreType.DMA,
            pltpu.SemaphoreType.DMA,
            pltpu.SemaphoreType.DMA,
        ],
    )
    def gather_rows(idx_hbm, table_hbm, out_hbm, idx_v, buf0, buf1,
                    gsem0, gsem1, ssem0, ssem1):
        wid = lax.axis_index("s") * _NC + lax.axis_index("c")
        base = wid * n_per_w
        pltpu.sync_copy(idx_hbm.at[wid], idx_v)
        bufs = (buf0, buf1)
        gsems = (gsem0, gsem1)
        ssems = (ssem0, ssem1)
        gathers = [None] * n_chunks
        scatters = [None] * n_chunks
        for g in range(n_chunks):
            if g >= 2:
                scatters[g - 2].wait()
            gathers[g] = pltpu.async_copy(
                table_hbm.at[idx_v.at[g]], bufs[g % 2], gsems[g % 2]
            )
            if g >= 1:
                gathers[g - 1].wait()
                scatters[g - 1] = pltpu.async_copy(
                    bufs[(g - 1) % 2],
                    out_hbm.at[pl.ds(base + (g - 1) * C, C)],
                    ssems[(g - 1) % 2],
                )
        gathers[n_chunks - 1].wait()
        scatters[n_chunks - 1] = pltpu.async_copy(
            bufs[(n_chunks - 1) % 2],
            out_hbm.at[pl.ds(base + (n_chunks - 1) * C, C)],
            ssems[(n_chunks - 1) % 2],
        )
        scatters[n_chunks - 2].wait()
        scatters[n_chunks - 1].wait()

    out = gather_rows(idx_flat, logits_table)
    return out.reshape(B, L, V)


# speedup vs baseline: 1.4826x; 1.0907x over previous
import functools

import jax
import jax.numpy as jnp
from jax import lax
from jax.experimental import pallas as pl
from jax.experimental.pallas import tpu as pltpu
from jax.experimental.pallas import tpu_sc as plsc

_NC = 2
_NS = 16
_NW = _NC * _NS


def kernel(idx, logits_table):
    B, L = idx.shape
    V, D = logits_table.shape
    N = B * L
    n_per_w = N // _NW
    C = 32
    n_chunks = n_per_w // C
    Vp = 1024

    idx_flat = idx.reshape(_NW, n_chunks, C).astype(jnp.int32)
    table_pad = jnp.pad(logits_table, ((0, Vp - V), (0, 0)))

    mesh = plsc.VectorSubcoreMesh(core_axis_name="c", subcore_axis_name="s")

    @functools.partial(
        pl.kernel,
        mesh=mesh,
        compiler_params=pltpu.CompilerParams(use_tc_tiling_on_sc=False),
        out_type=jax.ShapeDtypeStruct((N, D), jnp.float32),
        scratch_types=[
            pltpu.VMEM_SHARED((Vp, D), jnp.float32),
            pltpu.VMEM((n_chunks, C), jnp.int32),
            pltpu.VMEM((C, D), jnp.float32),
            pltpu.VMEM((C, D), jnp.float32),
            pltpu.SemaphoreType.DMA,
            pltpu.SemaphoreType.DMA,
            pltpu.SemaphoreType.DMA,
            pltpu.SemaphoreType.DMA,
        ],
    )
    def gather_rows(idx_hbm, table_hbm, out_hbm, table_sp, idx_v, buf0, buf1,
                    gsem0, gsem1, ssem0, ssem1):
        sid = lax.axis_index("s")
        wid = sid * _NC + lax.axis_index("c")
        base = wid * n_per_w
        rows_per_tile = Vp // _NS
        pltpu.sync_copy(
            table_hbm.at[pl.ds(sid * rows_per_tile, rows_per_tile)],
            table_sp.at[pl.ds(sid * rows_per_tile, rows_per_tile)],
        )
        pltpu.sync_copy(idx_hbm.at[wid], idx_v)
        plsc.subcore_barrier()
        bufs = (buf0, buf1)
        gsems = (gsem0, gsem1)
        ssems = (ssem0, ssem1)
        gathers = [None] * n_chunks
        scatters = [None] * n_chunks
        for g in range(n_chunks):
            if g >= 2:
                scatters[g - 2].wait()
            gathers[g] = pltpu.async_copy(
                table_sp.at[idx_v.at[g]], bufs[g % 2], gsems[g % 2]
            )
            if g >= 1:
                gathers[g - 1].wait()
                scatters[g - 1] = pltpu.async_copy(
                    bufs[(g - 1) % 2],
                    out_hbm.at[pl.ds(base + (g - 1) * C, C)],
                    ssems[(g - 1) % 2],
                )
        gathers[n_chunks - 1].wait()
        scatters[n_chunks - 1] = pltpu.async_copy(
            bufs[(n_chunks - 1) % 2],
            out_hbm.at[pl.ds(base + (n_chunks - 1) * C, C)],
            ssems[(n_chunks - 1) % 2],
        )
        scatters[n_chunks - 2].wait()
        scatters[n_chunks - 1].wait()

    out = gather_rows(idx_flat, table_pad)
    return out.reshape(B, L, V)


# speedup vs baseline: 1.4962x; 1.0092x over previous
import functools

import jax
import jax.numpy as jnp
from jax import lax
from jax.experimental import pallas as pl
from jax.experimental.pallas import tpu as pltpu
from jax.experimental.pallas import tpu_sc as plsc

_NC = 2
_NS = 16
_NW = _NC * _NS


def kernel(idx, logits_table):
    B, L = idx.shape
    V, D = logits_table.shape
    N = B * L
    n_per_w = N // _NW
    C = 32
    n_chunks = n_per_w // C

    idx_flat = idx.reshape(_NW, n_chunks, C).astype(jnp.int32)

    mesh = plsc.VectorSubcoreMesh(core_axis_name="c", subcore_axis_name="s")

    @functools.partial(
        pl.kernel,
        mesh=mesh,
        compiler_params=pltpu.CompilerParams(use_tc_tiling_on_sc=False),
        out_type=jax.ShapeDtypeStruct((N, D), jnp.float32),
        scratch_types=[
            pltpu.VMEM_SHARED((V, D), jnp.float32),
            pltpu.VMEM((n_chunks, C), jnp.int32),
            pltpu.VMEM((C, D), jnp.float32),
            pltpu.VMEM((C, D), jnp.float32),
            pltpu.SemaphoreType.DMA,
            pltpu.SemaphoreType.DMA,
            pltpu.SemaphoreType.DMA,
            pltpu.SemaphoreType.DMA,
        ],
    )
    def gather_rows(idx_hbm, table_hbm, out_hbm, table_sp, idx_v, buf0, buf1,
                    gsem0, gsem1, ssem0, ssem1):
        sid = lax.axis_index("s")
        wid = sid * _NC + lax.axis_index("c")
        base = wid * n_per_w
        stage_off = pl.multiple_of(jnp.minimum(sid * 64, V - 64), 8)
        pltpu.sync_copy(
            table_hbm.at[pl.ds(stage_off, 64)],
            table_sp.at[pl.ds(stage_off, 64)],
        )
        pltpu.sync_copy(idx_hbm.at[wid], idx_v)
        plsc.subcore_barrier()
        bufs = (buf0, buf1)
        gsems = (gsem0, gsem1)
        ssems = (ssem0, ssem1)
        gathers = [None] * n_chunks
        scatters = [None] * n_chunks
        for g in range(n_chunks):
            if g >= 2:
                scatters[g - 2].wait()
            gathers[g] = pltpu.async_copy(
                table_sp.at[idx_v.at[g]], bufs[g % 2], gsems[g % 2]
            )
            if g >= 1:
                gathers[g - 1].wait()
                scatters[g - 1] = pltpu.async_copy(
                    bufs[(g - 1) % 2],
                    out_hbm.at[pl.ds(base + (g - 1) * C, C)],
                    ssems[(g - 1) % 2],
                )
        gathers[n_chunks - 1].wait()
        scatters[n_chunks - 1] = pltpu.async_copy(
            bufs[(n_chunks - 1) % 2],
            out_hbm.at[pl.ds(base + (n_chunks - 1) * C, C)],
            ssems[(n_chunks - 1) % 2],
        )
        scatters[n_chunks - 2].wait()
        scatters[n_chunks - 1].wait()

    out = gather_rows(idx_flat, logits_table)
    return out.reshape(B, L, V)


# speedup vs baseline: 1.5002x; 1.0026x over previous
import functools

import jax
import jax.numpy as jnp
from jax import lax
from jax.experimental import pallas as pl
from jax.experimental.pallas import tpu as pltpu
from jax.experimental.pallas import tpu_sc as plsc

_NC = 2
_NS = 16
_NW = _NC * _NS


def kernel(idx, logits_table):
    B, L = idx.shape
    V, D = logits_table.shape
    b_per_w = B // _NW

    idx_i32 = idx.astype(jnp.int32)

    mesh = plsc.VectorSubcoreMesh(core_axis_name="c", subcore_axis_name="s")

    @functools.partial(
        pl.kernel,
        mesh=mesh,
        compiler_params=pltpu.CompilerParams(use_tc_tiling_on_sc=False),
        out_type=jax.ShapeDtypeStruct((B, L, D), jnp.float32),
        scratch_types=[
            pltpu.VMEM_SHARED((V, D), jnp.float32),
            pltpu.VMEM((b_per_w, L), jnp.int32),
            pltpu.VMEM((L, D), jnp.float32),
            pltpu.VMEM((L, D), jnp.float32),
            pltpu.SemaphoreType.DMA,
            pltpu.SemaphoreType.DMA,
            pltpu.SemaphoreType.DMA,
            pltpu.SemaphoreType.DMA,
        ],
    )
    def gather_rows(idx_hbm, table_hbm, out_hbm, table_sp, idx_v, buf0, buf1,
                    gsem0, gsem1, ssem0, ssem1):
        sid = lax.axis_index("s")
        wid = sid * _NC + lax.axis_index("c")
        base = wid * b_per_w
        stage_off = pl.multiple_of(jnp.minimum(sid * 64, V - 64), 8)
        pltpu.sync_copy(
            table_hbm.at[pl.ds(stage_off, 64)],
            table_sp.at[pl.ds(stage_off, 64)],
        )
        pltpu.sync_copy(idx_hbm.at[pl.ds(base, b_per_w)], idx_v)
        plsc.subcore_barrier()
        bufs = (buf0, buf1)
        gsems = (gsem0, gsem1)
        ssems = (ssem0, ssem1)
        gathers = [None] * b_per_w
        scatters = [None] * b_per_w
        for g in range(b_per_w):
            if g >= 2:
                scatters[g - 2].wait()
            gathers[g] = pltpu.async_copy(
                table_sp.at[idx_v.at[g]], bufs[g % 2], gsems[g % 2]
            )
            if g >= 1:
                gathers[g - 1].wait()
                scatters[g - 1] = pltpu.async_copy(
                    bufs[(g - 1) % 2],
                    out_hbm.at[base + g - 1],
                    ssems[(g - 1) % 2],
                )
        gathers[b_per_w - 1].wait()
        scatters[b_per_w - 1] = pltpu.async_copy(
            bufs[(b_per_w - 1) % 2],
            out_hbm.at[base + b_per_w - 1],
            ssems[(b_per_w - 1) % 2],
        )
        scatters[b_per_w - 2].wait()
        scatters[b_per_w - 1].wait()

    return gather_rows(idx_i32, logits_table)


# speedup vs baseline: 2.0628x; 1.3751x over previous
import functools

import jax
import jax.numpy as jnp
from jax import lax
from jax.experimental import pallas as pl
from jax.experimental.pallas import tpu as pltpu
from jax.experimental.pallas import tpu_sc as plsc

_NC = 2
_NS = 16
_NW = _NC * _NS


def kernel(idx, logits_table):
    B, L = idx.shape
    V, D = logits_table.shape
    b_per_w = B // _NW

    Dp = 1024
    idx_i32 = idx.reshape(_NW, b_per_w, L).astype(jnp.int32)
    table_p = jnp.pad(logits_table, ((0, 0), (0, Dp - D)))

    mesh = plsc.VectorSubcoreMesh(core_axis_name="c", subcore_axis_name="s")

    @functools.partial(
        pl.kernel,
        mesh=mesh,
        out_type=jax.ShapeDtypeStruct((B, L, Dp), jnp.float32),
        scratch_types=[
            pltpu.VMEM((b_per_w, L), jnp.int32),
            pltpu.VMEM((L, Dp), jnp.float32),
            pltpu.VMEM((L, Dp), jnp.float32),
            pltpu.SemaphoreType.DMA,
            pltpu.SemaphoreType.DMA,
            pltpu.SemaphoreType.DMA,
            pltpu.SemaphoreType.DMA,
        ],
    )
    def gather_rows(idx_hbm, table_hbm, out_hbm, idx_v, buf0, buf1,
                    gsem0, gsem1, ssem0, ssem1):
        sid = lax.axis_index("s")
        wid = sid * _NC + lax.axis_index("c")
        base = wid * b_per_w
        pltpu.sync_copy(idx_hbm.at[wid], idx_v)
        bufs = (buf0, buf1)
        gsems = (gsem0, gsem1)
        ssems = (ssem0, ssem1)
        gathers = [None] * b_per_w
        scatters = [None] * b_per_w
        for g in range(b_per_w):
            if g >= 2:
                scatters[g - 2].wait()
            gathers[g] = pltpu.async_copy(
                table_hbm.at[idx_v.at[g]], bufs[g % 2], gsems[g % 2]
            )
            if g >= 1:
                gathers[g - 1].wait()
                scatters[g - 1] = pltpu.async_copy(
                    bufs[(g - 1) % 2],
                    out_hbm.at[base + g - 1],
                    ssems[(g - 1) % 2],
                )
        gathers[b_per_w - 1].wait()
        scatters[b_per_w - 1] = pltpu.async_copy(
            bufs[(b_per_w - 1) % 2],
            out_hbm.at[base + b_per_w - 1],
            ssems[(b_per_w - 1) % 2],
        )
        scatters[b_per_w - 2].wait()
        scatters[b_per_w - 1].wait()

    out = gather_rows(idx_i32, table_p)
    return out[:, :, :D]
